# Initial kernel scaffold; baseline (speedup 1.0000x reference)
import jax
import jax.numpy as jnp
from jax.experimental import pallas as pl


def kernel(tokens, gamma, beta, W1, b1, W2, b2):
    raise NotImplementedError("write your pallas kernel here")


# speedup vs baseline: 1.9485x; 1.9485x over previous
import functools

import jax
import jax.numpy as jnp
from jax.experimental import pallas as pl
from jax.experimental.pallas import tpu as pltpu

_INV_SQRT2 = 0.7071067811865476


def _router_kernel(tok_ref, gamma_ref, beta_ref, w1_ref, b1_ref, w2_ref,
                   b2_ref, logits_ref, se_ref, ew_ref, xn_ref, acc_ref,
                   *, nh, e):
    h_idx = pl.program_id(1)

    @pl.when(h_idx == 0)
    def _layernorm():
        x = tok_ref[...]
        mu = jnp.mean(x, axis=-1, keepdims=True)
        xc = x - mu
        var = jnp.mean(xc * xc, axis=-1, keepdims=True)
        xn = xc * jax.lax.rsqrt(var + 1e-5) * gamma_ref[...] + beta_ref[...]
        xn_ref[...] = xn.astype(jnp.bfloat16)

    hblk = jnp.dot(xn_ref[...], w1_ref[...],
                   preferred_element_type=jnp.float32)
    hblk = hblk + b1_ref[...]
    g = hblk * 0.5 * (1.0 + jax.lax.erf(hblk * _INV_SQRT2))
    part = jnp.dot(g.astype(jnp.bfloat16), w2_ref[...],
                   preferred_element_type=jnp.float32)

    @pl.when(h_idx == 0)
    def _init():
        acc_ref[...] = part

    @pl.when(h_idx > 0)
    def _accum():
        acc_ref[...] += part

    @pl.when(h_idx == nh - 1)
    def _finalize():
        logits = acc_ref[...] + b2_ref[...]
        logits_ref[...] = logits
        col = jax.lax.broadcasted_iota(jnp.int32, logits.shape, 1)
        m1 = jnp.max(logits, axis=1, keepdims=True)
        i1 = jnp.min(jnp.where(logits == m1, col, e), axis=1, keepdims=True)
        masked = jnp.where(col == i1, -jnp.inf, logits)
        m2 = jnp.max(masked, axis=1, keepdims=True)
        i2 = jnp.min(jnp.where(masked == m2, col, e), axis=1, keepdims=True)
        t = jnp.exp(m2 - m1)
        s = 1.0 + t
        wa = 1.0 / s
        wb = t / s
        ew_ref[...] = jnp.where(col == i1, wa,
                                jnp.where(col == i2, wb, 0.0))
        se_ref[...] = jnp.concatenate([i1, i2], axis=1)


def kernel(tokens, gamma, beta, W1, b1, W2, b2):
    n, d = tokens.shape
    h = W1.shape[1]
    e = W2.shape[1]
    bm = min(512, n)
    bh = min(1024, h)
    grid = (n // bm, h // bh)

    out = pl.pallas_call(
        functools.partial(_router_kernel, nh=grid[1], e=e),
        grid=grid,
        in_specs=[
            pl.BlockSpec((bm, d), lambda m, hh: (m, 0)),
            pl.BlockSpec((1, d), lambda m, hh: (0, 0)),
            pl.BlockSpec((1, d), lambda m, hh: (0, 0)),
            pl.BlockSpec((d, bh), lambda m, hh: (0, hh)),
            pl.BlockSpec((1, bh), lambda m, hh: (0, hh)),
            pl.BlockSpec((bh, e), lambda m, hh: (hh, 0)),
            pl.BlockSpec((1, e), lambda m, hh: (0, 0)),
        ],
        out_specs=[
            pl.BlockSpec((bm, e), lambda m, hh: (m, 0)),
            pl.BlockSpec((bm, 2), lambda m, hh: (m, 0)),
            pl.BlockSpec((bm, e), lambda m, hh: (m, 0)),
        ],
        out_shape=[
            jax.ShapeDtypeStruct((n, e), jnp.float32),
            jax.ShapeDtypeStruct((n, 2), jnp.int32),
            jax.ShapeDtypeStruct((n, e), jnp.float32),
        ],
        scratch_shapes=[
            pltpu.VMEM((bm, d), jnp.bfloat16),
            pltpu.VMEM((bm, e), jnp.float32),
        ],
    )(tokens, gamma.reshape(1, d), beta.reshape(1, d),
      W1.astype(jnp.bfloat16), b1.reshape(1, h),
      W2.astype(jnp.bfloat16), b2.reshape(1, e))
    return (out[0], out[1], out[2])
